# Initial kernel scaffold; baseline (speedup 1.0000x reference)
import jax
import jax.numpy as jnp
from jax.experimental import pallas as pl


def kernel(atom_type, degree, formal_charge, hybrid, num_h, chirality, bond_counts, scalar_base, atom_tab, deg_tab, charge_tab, hyb_tab, h_tab, chir_tab, bond_tab, W, b):
    raise NotImplementedError("write your pallas kernel here")


# speedup vs baseline: 2.0294x; 2.0294x over previous
import functools

import jax
import jax.numpy as jnp
from jax import lax
from jax.experimental import pallas as pl
from jax.experimental.pallas import tpu as pltpu
from jax.experimental.pallas import tpu_sc as plsc

D = 128
B = 64
NC = 2
NS = 16
NW = NC * NS
L = 16
NROWS = 46 * 3 + 6 * 11 + 5 * 4 * 5 + 5 * 5 * 5
OFF_B = 46 * 3
OFF_C = OFF_B + 6 * 11
OFF_D = OFF_C + 5 * 4 * 5


def _featurize_body(t_h, wc_h,
                    a_h, d_h, c_h, h_h, n_h, ch_h,
                    b0_h, b1_h, b2_h, b3_h, s0_h, s1_h, s2_h,
                    out_h,
                    t_v, wc_v,
                    ai_v, di_v, ci_v, hi_v, ni_v, chi_v, bc_v, sb_v,
                    o_v,
                    is0, is1, os0, os1,
                    *, n_atoms, cnt_main):
  wid = lax.axis_index("s") * NC + lax.axis_index("c")
  base = wid * cnt_main
  cnt = jnp.minimum(cnt_main, n_atoms - base)
  trips = cnt // B
  pairs = trips // 2
  main_batches = pairs * 2
  isems = (is0, is1)
  osems = (os0, os1)

  pltpu.sync_copy(t_h, t_v)
  pltpu.sync_copy(wc_h, wc_v)

  lane = jnp.arange(L, dtype=jnp.int32)

  def issue_inputs(buf, off):
    ib = pl.ds(buf * B, B)
    pltpu.async_copy(a_h.at[pl.ds(off, B)], ai_v.at[ib], isems[buf])
    pltpu.async_copy(d_h.at[pl.ds(off, B)], di_v.at[ib], isems[buf])
    pltpu.async_copy(c_h.at[pl.ds(off, B)], ci_v.at[ib], isems[buf])
    pltpu.async_copy(h_h.at[pl.ds(off, B)], hi_v.at[ib], isems[buf])
    pltpu.async_copy(n_h.at[pl.ds(off, B)], ni_v.at[ib], isems[buf])
    pltpu.async_copy(ch_h.at[pl.ds(off, B)], chi_v.at[ib], isems[buf])
    src = pl.ds(off, B)
    pltpu.async_copy(b0_h.at[src], bc_v.at[pl.ds((buf * 4) * B, B)],
                     isems[buf])
    pltpu.async_copy(b1_h.at[src], bc_v.at[pl.ds((buf * 4 + 1) * B, B)],
                     isems[buf])
    pltpu.async_copy(b2_h.at[src], bc_v.at[pl.ds((buf * 4 + 2) * B, B)],
                     isems[buf])
    pltpu.async_copy(b3_h.at[src], bc_v.at[pl.ds((buf * 4 + 3) * B, B)],
                     isems[buf])
    pltpu.async_copy(s0_h.at[src], sb_v.at[pl.ds((buf * 3) * B, B)],
                     isems[buf])
    pltpu.async_copy(s1_h.at[src], sb_v.at[pl.ds((buf * 3 + 1) * B, B)],
                     isems[buf])
    pltpu.async_copy(s2_h.at[src], sb_v.at[pl.ds((buf * 3 + 2) * B, B)],
                     isems[buf])

  def drain_inputs(buf):
    ib = pl.ds(buf * B, B)
    src = pl.ds(0, B)
    pltpu.make_async_copy(a_h.at[src], ai_v.at[ib], isems[buf]).wait()
    pltpu.make_async_copy(d_h.at[src], di_v.at[ib], isems[buf]).wait()
    pltpu.make_async_copy(c_h.at[src], ci_v.at[ib], isems[buf]).wait()
    pltpu.make_async_copy(h_h.at[src], hi_v.at[ib], isems[buf]).wait()
    pltpu.make_async_copy(n_h.at[src], ni_v.at[ib], isems[buf]).wait()
    pltpu.make_async_copy(ch_h.at[src], chi_v.at[ib], isems[buf]).wait()
    pltpu.make_async_copy(b0_h.at[src], bc_v.at[pl.ds((buf * 4) * B, B)],
                          isems[buf]).wait()
    pltpu.make_async_copy(b1_h.at[src], bc_v.at[pl.ds((buf * 4 + 1) * B, B)],
                          isems[buf]).wait()
    pltpu.make_async_copy(b2_h.at[src], bc_v.at[pl.ds((buf * 4 + 2) * B, B)],
                          isems[buf]).wait()
    pltpu.make_async_copy(b3_h.at[src], bc_v.at[pl.ds((buf * 4 + 3) * B, B)],
                          isems[buf]).wait()
    pltpu.make_async_copy(s0_h.at[src], sb_v.at[pl.ds((buf * 3) * B, B)],
                          isems[buf]).wait()
    pltpu.make_async_copy(s1_h.at[src], sb_v.at[pl.ds((buf * 3 + 1) * B, B)],
                          isems[buf]).wait()
    pltpu.make_async_copy(s2_h.at[src], sb_v.at[pl.ds((buf * 3 + 2) * B, B)],
                          isems[buf]).wait()

  def compute_atoms(buf, nmac):
    @plsc.parallel_loop(0, nmac)
    def mbody(m):
      sl = pl.ds(buf * B + m * L, L)

      def bsl(j):
        return pl.ds((buf * 4 + j) * B + m * L, L)

      def ssl(j):
        return pl.ds((buf * 3 + j) * B + m * L, L)

      fa = ai_v[sl] * 3 + hi_v[sl]
      fb = di_v[sl] * 11 + ci_v[sl] + OFF_B
      fc = (ni_v[sl] * 4 + chi_v[sl]) * 5 + bc_v[bsl(0)] + OFF_C
      fd = (bc_v[bsl(1)] * 5 + bc_v[bsl(2)]) * 5 + bc_v[bsl(3)] + OFF_D
      s0v = sb_v[ssl(0)]
      s1v = sb_v[ssl(1)]
      s2v = sb_v[ssl(2)]
      arow = lane + m * L
      ob = o_v.at[buf]
      for d in range(D):
        dsp = jnp.full((L,), d, jnp.int32)
        w0b = wc_v[0, d, :]
        w1b = wc_v[1, d, :]
        w2b = wc_v[2, d, :]
        ga = plsc.load_gather(t_v, [fa, dsp])
        gb = plsc.load_gather(t_v, [fb, dsp])
        gc = plsc.load_gather(t_v, [fc, dsp])
        gd = plsc.load_gather(t_v, [fd, dsp])
        acc = (ga + gb) + (gc + gd)
        acc = acc + (s0v * w0b + s1v * w1b) + s2v * w2b
        plsc.store_scatter(ob, [arow, dsp], acc)

  @pl.when(main_batches > 0)
  def _():
    issue_inputs(0, base)

  @pl.when(main_batches > 1)
  def _():
    issue_inputs(1, base + B)

  def pair_body(p, _):
    for buf in range(2):
      g = p * 2 + buf
      off = base + g * B
      @pl.when(p > 0)
      def _():
        pltpu.make_async_copy(o_v.at[buf], out_h.at[pl.ds(base, B)],
                              osems[buf]).wait()
      drain_inputs(buf)
      compute_atoms(buf, B // L)
      pltpu.async_copy(o_v.at[buf], out_h.at[pl.ds(off, B)], osems[buf])
      @pl.when(g + 2 < main_batches)
      def _():
        issue_inputs(buf, off + 2 * B)
    return 0

  lax.fori_loop(0, pairs, pair_body, 0)

  @pl.when(pairs > 0)
  def _():
    pltpu.make_async_copy(o_v.at[0], out_h.at[pl.ds(base, B)],
                          osems[0]).wait()
    pltpu.make_async_copy(o_v.at[1], out_h.at[pl.ds(base, B)],
                          osems[1]).wait()

  tail16 = (cnt - main_batches * B) // L

  def tail_body(t, _):
    off = base + main_batches * B + t * L
    pltpu.sync_copy(a_h.at[pl.ds(off, L)], ai_v.at[pl.ds(0, L)])
    pltpu.sync_copy(d_h.at[pl.ds(off, L)], di_v.at[pl.ds(0, L)])
    pltpu.sync_copy(c_h.at[pl.ds(off, L)], ci_v.at[pl.ds(0, L)])
    pltpu.sync_copy(h_h.at[pl.ds(off, L)], hi_v.at[pl.ds(0, L)])
    pltpu.sync_copy(n_h.at[pl.ds(off, L)], ni_v.at[pl.ds(0, L)])
    pltpu.sync_copy(ch_h.at[pl.ds(off, L)], chi_v.at[pl.ds(0, L)])
    pltpu.sync_copy(b0_h.at[pl.ds(off, L)], bc_v.at[pl.ds(0, L)])
    pltpu.sync_copy(b1_h.at[pl.ds(off, L)], bc_v.at[pl.ds(B, L)])
    pltpu.sync_copy(b2_h.at[pl.ds(off, L)], bc_v.at[pl.ds(2 * B, L)])
    pltpu.sync_copy(b3_h.at[pl.ds(off, L)], bc_v.at[pl.ds(3 * B, L)])
    pltpu.sync_copy(s0_h.at[pl.ds(off, L)], sb_v.at[pl.ds(0, L)])
    pltpu.sync_copy(s1_h.at[pl.ds(off, L)], sb_v.at[pl.ds(B, L)])
    pltpu.sync_copy(s2_h.at[pl.ds(off, L)], sb_v.at[pl.ds(2 * B, L)])
    compute_atoms(0, 1)
    pltpu.sync_copy(o_v.at[0, pl.ds(0, L)], out_h.at[pl.ds(off, L)])
    return 0

  lax.fori_loop(0, tail16, tail_body, 0)


@functools.partial(jax.jit, static_argnames=("n_atoms",))
def _featurize(t, wc, a, d, c, h, n, ch,
               b0, b1, b2, b3, s0, s1, s2, *, n_atoms):
  cnt_main = ((n_atoms + NW - 1) // NW + L - 1) // L * L
  mesh = plsc.VectorSubcoreMesh(core_axis_name="c", subcore_axis_name="s")
  body = functools.partial(_featurize_body, n_atoms=n_atoms,
                           cnt_main=cnt_main)
  return pl.kernel(
      body,
      out_type=jax.ShapeDtypeStruct((n_atoms, D), jnp.float32),
      mesh=mesh,
      compiler_params=pltpu.CompilerParams(needs_layout_passes=False),
      scratch_types=[
          pltpu.VMEM((NROWS, D), jnp.float32),
          pltpu.VMEM((3, D, L), jnp.float32),
          pltpu.VMEM((2 * B,), jnp.int32),
          pltpu.VMEM((2 * B,), jnp.int32),
          pltpu.VMEM((2 * B,), jnp.int32),
          pltpu.VMEM((2 * B,), jnp.int32),
          pltpu.VMEM((2 * B,), jnp.int32),
          pltpu.VMEM((2 * B,), jnp.int32),
          pltpu.VMEM((8 * B,), jnp.int32),
          pltpu.VMEM((6 * B,), jnp.float32),
          pltpu.VMEM((2, B, D), jnp.float32),
          pltpu.SemaphoreType.DMA,
          pltpu.SemaphoreType.DMA,
          pltpu.SemaphoreType.DMA,
          pltpu.SemaphoreType.DMA,
      ],
  )(t, wc, a, d, c, h, n, ch, b0, b1, b2, b3, s0, s1, s2)


def kernel(atom_type, degree, formal_charge, hybrid, num_h, chirality,
           bond_counts, scalar_base, atom_tab, deg_tab, charge_tab, hyb_tab,
           h_tab, chir_tab, bond_tab, W, b):
  n_atoms = atom_type.shape[0]
  cnt5 = jnp.arange(5, dtype=jnp.float32)
  gate = (cnt5 > 0).astype(jnp.float32)
  g = [bond_tab * gate[:, None] + (cnt5 / 4.0)[:, None] * W[:, 3 + j][None, :]
       for j in range(4)]
  ta = (atom_tab[:, None, :] + hyb_tab[None, :, :]
        + b[None, None, :]).reshape(46 * 3, D)
  tb = (deg_tab[:, None, :] + charge_tab[None, :, :]).reshape(6 * 11, D)
  tc = (h_tab[:, None, None, :] + chir_tab[None, :, None, :]
        + g[0][None, None, :, :]).reshape(5 * 4 * 5, D)
  td = (g[1][:, None, None, :] + g[2][None, :, None, :]
        + g[3][None, None, :, :]).reshape(5 * 5 * 5, D)
  t = jnp.concatenate([ta, tb, tc, td], axis=0)
  wc = jnp.broadcast_to(jnp.transpose(W)[:3, :, None], (3, D, L))
  a = atom_type.astype(jnp.int32)
  d = degree.astype(jnp.int32)
  c = formal_charge.astype(jnp.int32)
  h = hybrid.astype(jnp.int32)
  n = num_h.astype(jnp.int32)
  ch = chirality.astype(jnp.int32)
  bc = bond_counts.astype(jnp.int32)
  sb = scalar_base
  return _featurize(t, wc, a, d, c, h, n, ch,
                    bc[:, 0], bc[:, 1], bc[:, 2], bc[:, 3],
                    sb[:, 0], sb[:, 1], sb[:, 2], n_atoms=n_atoms)


# speedup vs baseline: 25.7333x; 12.6803x over previous
import functools

import jax
import jax.numpy as jnp
from jax import lax
from jax.experimental import pallas as pl
from jax.experimental.pallas import tpu as pltpu
from jax.experimental.pallas import tpu_sc as plsc

D = 128
B = 128
NC = 2
NS = 16
NW = NC * NS
L = 16


def _featurize_body(ta_h, tb_h, tc_h, td_h, wc_h,
                    a_h, d_h, c_h, h_h, n_h, ch_h,
                    b0_h, b1_h, b2_h, b3_h, s0_h, s1_h, s2_h,
                    out_h,
                    ta_v, tb_v, tc_v, td_v, wc_v,
                    ai_v, di_v, ci_v, hi_v, ni_v, chi_v, bc_v, sb_v,
                    o_v,
                    is0, is1, os0, os1,
                    *, n_atoms, cnt_main):
  wid = lax.axis_index("s") * NC + lax.axis_index("c")
  base = wid * cnt_main
  cnt = jnp.minimum(cnt_main, n_atoms - base)
  trips = cnt // B
  pairs = trips // 2
  main_batches = pairs * 2
  isems = (is0, is1)
  osems = (os0, os1)

  pltpu.sync_copy(ta_h, ta_v)
  pltpu.sync_copy(tb_h, tb_v)
  pltpu.sync_copy(tc_h, tc_v)
  pltpu.sync_copy(td_h, td_v)
  pltpu.sync_copy(wc_h, wc_v)

  wcs = [[wc_v[j, pl.ds(k * L, L)] for k in range(8)] for j in range(3)]

  def issue_inputs(buf, off):
    pltpu.async_copy(a_h.at[pl.ds(off, B)], ai_v.at[buf], isems[buf])
    pltpu.async_copy(d_h.at[pl.ds(off, B)], di_v.at[buf], isems[buf])
    pltpu.async_copy(c_h.at[pl.ds(off, B)], ci_v.at[buf], isems[buf])
    pltpu.async_copy(h_h.at[pl.ds(off, B)], hi_v.at[buf], isems[buf])
    pltpu.async_copy(n_h.at[pl.ds(off, B)], ni_v.at[buf], isems[buf])
    pltpu.async_copy(ch_h.at[pl.ds(off, B)], chi_v.at[buf], isems[buf])
    pltpu.async_copy(b0_h.at[pl.ds(off, B)], bc_v.at[buf, 0], isems[buf])
    pltpu.async_copy(b1_h.at[pl.ds(off, B)], bc_v.at[buf, 1], isems[buf])
    pltpu.async_copy(b2_h.at[pl.ds(off, B)], bc_v.at[buf, 2], isems[buf])
    pltpu.async_copy(b3_h.at[pl.ds(off, B)], bc_v.at[buf, 3], isems[buf])
    pltpu.async_copy(s0_h.at[pl.ds(off, B)], sb_v.at[buf, 0], isems[buf])
    pltpu.async_copy(s1_h.at[pl.ds(off, B)], sb_v.at[buf, 1], isems[buf])
    pltpu.async_copy(s2_h.at[pl.ds(off, B)], sb_v.at[buf, 2], isems[buf])

  def drain_inputs(buf):
    pltpu.make_async_copy(a_h.at[pl.ds(0, B)], ai_v.at[buf], isems[buf]).wait()
    pltpu.make_async_copy(d_h.at[pl.ds(0, B)], di_v.at[buf], isems[buf]).wait()
    pltpu.make_async_copy(c_h.at[pl.ds(0, B)], ci_v.at[buf], isems[buf]).wait()
    pltpu.make_async_copy(h_h.at[pl.ds(0, B)], hi_v.at[buf], isems[buf]).wait()
    pltpu.make_async_copy(n_h.at[pl.ds(0, B)], ni_v.at[buf], isems[buf]).wait()
    pltpu.make_async_copy(ch_h.at[pl.ds(0, B)], chi_v.at[buf],
                          isems[buf]).wait()
    pltpu.make_async_copy(b0_h.at[pl.ds(0, B)], bc_v.at[buf, 0],
                          isems[buf]).wait()
    pltpu.make_async_copy(b1_h.at[pl.ds(0, B)], bc_v.at[buf, 1],
                          isems[buf]).wait()
    pltpu.make_async_copy(b2_h.at[pl.ds(0, B)], bc_v.at[buf, 2],
                          isems[buf]).wait()
    pltpu.make_async_copy(b3_h.at[pl.ds(0, B)], bc_v.at[buf, 3],
                          isems[buf]).wait()
    pltpu.make_async_copy(s0_h.at[pl.ds(0, B)], sb_v.at[buf, 0],
                          isems[buf]).wait()
    pltpu.make_async_copy(s1_h.at[pl.ds(0, B)], sb_v.at[buf, 1],
                          isems[buf]).wait()
    pltpu.make_async_copy(s2_h.at[pl.ds(0, B)], sb_v.at[buf, 2],
                          isems[buf]).wait()

  def compute_atoms(buf, nmac):
    @plsc.parallel_loop(0, nmac)
    def mbody(m):
      sl = pl.ds(m * L, L)
      iav = ai_v[buf, sl] * 6 + di_v[buf, sl]
      ibv = (ci_v[buf, sl] * 3 + hi_v[buf, sl]) * 5 + ni_v[buf, sl]
      icv = chi_v[buf, sl] * 5 + bc_v[buf, 0, sl]
      idv = (bc_v[buf, 1, sl] * 5 + bc_v[buf, 2, sl]) * 5 + bc_v[buf, 3, sl]
      s0v = sb_v[buf, 0, sl]
      s1v = sb_v[buf, 1, sl]
      s2v = sb_v[buf, 2, sl]
      for j in range(L):
        iA = iav[j]
        iB = ibv[j]
        iC = icv[j]
        iD = idv[j]
        s0 = s0v[j]
        s1 = s1v[j]
        s2 = s2v[j]
        i = m * L + j
        accs = []
        for k in range(8):
          slk = pl.ds(k * L, L)
          acc = ta_v[iA, slk] + tb_v[iB, slk] + tc_v[iC, slk] + td_v[iD, slk]
          acc = acc + s0 * wcs[0][k] + s1 * wcs[1][k] + s2 * wcs[2][k]
          accs.append(acc)
        for k in range(8):
          o_v[buf, i, pl.ds(k * L, L)] = accs[k]

  @pl.when(main_batches > 0)
  def _():
    issue_inputs(0, base)

  @pl.when(main_batches > 1)
  def _():
    issue_inputs(1, base + B)

  def pair_body(p, _):
    for buf in range(2):
      g = p * 2 + buf
      off = base + g * B
      @pl.when(p > 0)
      def _():
        pltpu.make_async_copy(o_v.at[buf], out_h.at[pl.ds(base, B)],
                              osems[buf]).wait()
      drain_inputs(buf)
      compute_atoms(buf, B // L)
      pltpu.async_copy(o_v.at[buf], out_h.at[pl.ds(off, B)], osems[buf])
      @pl.when(g + 2 < main_batches)
      def _():
        issue_inputs(buf, off + 2 * B)
    return 0

  lax.fori_loop(0, pairs, pair_body, 0)

  @pl.when(pairs > 0)
  def _():
    pltpu.make_async_copy(o_v.at[0], out_h.at[pl.ds(base, B)], osems[0]).wait()
    pltpu.make_async_copy(o_v.at[1], out_h.at[pl.ds(base, B)], osems[1]).wait()

  tail16 = (cnt - main_batches * B) // L

  def tail_body(t, _):
    off = base + main_batches * B + t * L
    pltpu.sync_copy(a_h.at[pl.ds(off, L)], ai_v.at[0, pl.ds(0, L)])
    pltpu.sync_copy(d_h.at[pl.ds(off, L)], di_v.at[0, pl.ds(0, L)])
    pltpu.sync_copy(c_h.at[pl.ds(off, L)], ci_v.at[0, pl.ds(0, L)])
    pltpu.sync_copy(h_h.at[pl.ds(off, L)], hi_v.at[0, pl.ds(0, L)])
    pltpu.sync_copy(n_h.at[pl.ds(off, L)], ni_v.at[0, pl.ds(0, L)])
    pltpu.sync_copy(ch_h.at[pl.ds(off, L)], chi_v.at[0, pl.ds(0, L)])
    pltpu.sync_copy(b0_h.at[pl.ds(off, L)], bc_v.at[0, 0, pl.ds(0, L)])
    pltpu.sync_copy(b1_h.at[pl.ds(off, L)], bc_v.at[0, 1, pl.ds(0, L)])
    pltpu.sync_copy(b2_h.at[pl.ds(off, L)], bc_v.at[0, 2, pl.ds(0, L)])
    pltpu.sync_copy(b3_h.at[pl.ds(off, L)], bc_v.at[0, 3, pl.ds(0, L)])
    pltpu.sync_copy(s0_h.at[pl.ds(off, L)], sb_v.at[0, 0, pl.ds(0, L)])
    pltpu.sync_copy(s1_h.at[pl.ds(off, L)], sb_v.at[0, 1, pl.ds(0, L)])
    pltpu.sync_copy(s2_h.at[pl.ds(off, L)], sb_v.at[0, 2, pl.ds(0, L)])
    compute_atoms(0, 1)
    pltpu.sync_copy(o_v.at[0, pl.ds(0, L)], out_h.at[pl.ds(off, L)])
    return 0

  lax.fori_loop(0, tail16, tail_body, 0)


@functools.partial(jax.jit, static_argnames=("n_atoms",))
def _featurize(ta, tb, tc, td, wc, a, d, c, h, n, ch,
               b0, b1, b2, b3, s0, s1, s2, *, n_atoms):
  cnt_main = ((n_atoms + NW - 1) // NW + L - 1) // L * L
  mesh = plsc.VectorSubcoreMesh(core_axis_name="c", subcore_axis_name="s")
  body = functools.partial(_featurize_body, n_atoms=n_atoms,
                           cnt_main=cnt_main)
  return pl.kernel(
      body,
      out_type=jax.ShapeDtypeStruct((n_atoms, D), jnp.float32),
      mesh=mesh,
      scratch_types=[
          pltpu.VMEM((276, D), jnp.float32),
          pltpu.VMEM((165, D), jnp.float32),
          pltpu.VMEM((20, D), jnp.float32),
          pltpu.VMEM((125, D), jnp.float32),
          pltpu.VMEM((3, D), jnp.float32),
          pltpu.VMEM((2, B), jnp.int32),
          pltpu.VMEM((2, B), jnp.int32),
          pltpu.VMEM((2, B), jnp.int32),
          pltpu.VMEM((2, B), jnp.int32),
          pltpu.VMEM((2, B), jnp.int32),
          pltpu.VMEM((2, B), jnp.int32),
          pltpu.VMEM((2, 4, B), jnp.int32),
          pltpu.VMEM((2, 3, B), jnp.float32),
          pltpu.VMEM((2, B, D), jnp.float32),
          pltpu.SemaphoreType.DMA,
          pltpu.SemaphoreType.DMA,
          pltpu.SemaphoreType.DMA,
          pltpu.SemaphoreType.DMA,
      ],
  )(ta, tb, tc, td, wc, a, d, c, h, n, ch, b0, b1, b2, b3, s0, s1, s2)


def kernel(atom_type, degree, formal_charge, hybrid, num_h, chirality,
           bond_counts, scalar_base, atom_tab, deg_tab, charge_tab, hyb_tab,
           h_tab, chir_tab, bond_tab, W, b):
  n_atoms = atom_type.shape[0]
  cnt5 = jnp.arange(5, dtype=jnp.float32)
  gate = (cnt5 > 0).astype(jnp.float32)
  g = [bond_tab * gate[:, None] + (cnt5 / 4.0)[:, None] * W[:, 3 + j][None, :]
       for j in range(4)]
  ta = (atom_tab[:, None, :] + deg_tab[None, :, :]
        + b[None, None, :]).reshape(46 * 6, D)
  tb = (charge_tab[:, None, None, :] + hyb_tab[None, :, None, :]
        + h_tab[None, None, :, :]).reshape(11 * 3 * 5, D)
  tc = (chir_tab[:, None, :] + g[0][None, :, :]).reshape(4 * 5, D)
  td = (g[1][:, None, None, :] + g[2][None, :, None, :]
        + g[3][None, None, :, :]).reshape(5 * 5 * 5, D)
  wc = jnp.transpose(W)[:3]
  a = atom_type.astype(jnp.int32)
  d = degree.astype(jnp.int32)
  c = formal_charge.astype(jnp.int32)
  h = hybrid.astype(jnp.int32)
  n = num_h.astype(jnp.int32)
  ch = chirality.astype(jnp.int32)
  bc = bond_counts.astype(jnp.int32)
  sb = scalar_base
  return _featurize(ta, tb, tc, td, wc, a, d, c, h, n, ch,
                    bc[:, 0], bc[:, 1], bc[:, 2], bc[:, 3],
                    sb[:, 0], sb[:, 1], sb[:, 2], n_atoms=n_atoms)


# speedup vs baseline: 31.3481x; 1.2182x over previous
import functools

import jax
import jax.numpy as jnp
import numpy as np
from jax import lax
from jax.experimental import pallas as pl
from jax.experimental.pallas import tpu as pltpu
from jax.experimental.pallas import tpu_sc as plsc

D = 128
DW = D // 2
B = 128
NC = 2
NS = 16
NW = NC * NS
L = 16

_PERM = np.concatenate([
    np.stack([np.arange(g * 32, g * 32 + 16),
              np.arange(g * 32 + 16, g * 32 + 32)], axis=1).reshape(-1)
    for g in range(4)])


def _featurize_body(ta_h, tb_h, tc_h, td_h, wc_h,
                    a_h, d_h, c_h, h_h, n_h, ch_h,
                    b0_h, b1_h, b2_h, b3_h, s0_h, s1_h, s2_h,
                    out_h,
                    ta_v, tb_v, tc_v, td_v, wc_v,
                    ai_v, di_v, ci_v, hi_v, ni_v, chi_v, bc_v, sb_v,
                    o_v,
                    is0, is1, os0, os1,
                    *, n_atoms, cnt_main):
  wid = lax.axis_index("s") * NC + lax.axis_index("c")
  base = wid * cnt_main
  cnt = jnp.minimum(cnt_main, n_atoms - base)
  trips = cnt // B
  pairs = trips // 2
  main_batches = pairs * 2
  isems = (is0, is1)
  osems = (os0, os1)

  pltpu.sync_copy(ta_h, ta_v)
  pltpu.sync_copy(tb_h, tb_v)
  pltpu.sync_copy(tc_h, tc_v)
  pltpu.sync_copy(td_h, td_v)
  pltpu.sync_copy(wc_h, wc_v)

  wcs = [[wc_v[j, pl.ds(k * L, L)] for k in range(8)] for j in range(3)]

  def issue_inputs(buf, off):
    ib = pl.ds(buf * B, B)
    src = pl.ds(off, B)
    pltpu.async_copy(a_h.at[src], ai_v.at[ib], isems[buf])
    pltpu.async_copy(d_h.at[src], di_v.at[ib], isems[buf])
    pltpu.async_copy(c_h.at[src], ci_v.at[ib], isems[buf])
    pltpu.async_copy(h_h.at[src], hi_v.at[ib], isems[buf])
    pltpu.async_copy(n_h.at[src], ni_v.at[ib], isems[buf])
    pltpu.async_copy(ch_h.at[src], chi_v.at[ib], isems[buf])
    pltpu.async_copy(b0_h.at[src], bc_v.at[pl.ds((buf * 4) * B, B)],
                     isems[buf])
    pltpu.async_copy(b1_h.at[src], bc_v.at[pl.ds((buf * 4 + 1) * B, B)],
                     isems[buf])
    pltpu.async_copy(b2_h.at[src], bc_v.at[pl.ds((buf * 4 + 2) * B, B)],
                     isems[buf])
    pltpu.async_copy(b3_h.at[src], bc_v.at[pl.ds((buf * 4 + 3) * B, B)],
                     isems[buf])
    pltpu.async_copy(s0_h.at[src], sb_v.at[pl.ds((buf * 3) * B, B)],
                     isems[buf])
    pltpu.async_copy(s1_h.at[src], sb_v.at[pl.ds((buf * 3 + 1) * B, B)],
                     isems[buf])
    pltpu.async_copy(s2_h.at[src], sb_v.at[pl.ds((buf * 3 + 2) * B, B)],
                     isems[buf])

  def drain_inputs(buf):
    ib = pl.ds(buf * B, B)
    src = pl.ds(0, B)
    pltpu.make_async_copy(a_h.at[src], ai_v.at[ib], isems[buf]).wait()
    pltpu.make_async_copy(d_h.at[src], di_v.at[ib], isems[buf]).wait()
    pltpu.make_async_copy(c_h.at[src], ci_v.at[ib], isems[buf]).wait()
    pltpu.make_async_copy(h_h.at[src], hi_v.at[ib], isems[buf]).wait()
    pltpu.make_async_copy(n_h.at[src], ni_v.at[ib], isems[buf]).wait()
    pltpu.make_async_copy(ch_h.at[src], chi_v.at[ib], isems[buf]).wait()
    pltpu.make_async_copy(b0_h.at[src], bc_v.at[pl.ds((buf * 4) * B, B)],
                          isems[buf]).wait()
    pltpu.make_async_copy(b1_h.at[src], bc_v.at[pl.ds((buf * 4 + 1) * B, B)],
                          isems[buf]).wait()
    pltpu.make_async_copy(b2_h.at[src], bc_v.at[pl.ds((buf * 4 + 2) * B, B)],
                          isems[buf]).wait()
    pltpu.make_async_copy(b3_h.at[src], bc_v.at[pl.ds((buf * 4 + 3) * B, B)],
                          isems[buf]).wait()
    pltpu.make_async_copy(s0_h.at[src], sb_v.at[pl.ds((buf * 3) * B, B)],
                          isems[buf]).wait()
    pltpu.make_async_copy(s1_h.at[src], sb_v.at[pl.ds((buf * 3 + 1) * B, B)],
                          isems[buf]).wait()
    pltpu.make_async_copy(s2_h.at[src], sb_v.at[pl.ds((buf * 3 + 2) * B, B)],
                          isems[buf]).wait()

  def compute_atoms(buf, nmac):
    @plsc.parallel_loop(0, nmac)
    def mbody(m):
      sl = pl.ds(buf * B + m * L, L)

      def bsl(j):
        return pl.ds((buf * 4 + j) * B + m * L, L)

      def ssl(j):
        return pl.ds((buf * 3 + j) * B + m * L, L)

      iav = ai_v[sl] * 6 + di_v[sl]
      ibv = (ci_v[sl] * 3 + hi_v[sl]) * 5 + ni_v[sl]
      icv = chi_v[sl] * 5 + bc_v[bsl(0)]
      idv = (bc_v[bsl(1)] * 5 + bc_v[bsl(2)]) * 5 + bc_v[bsl(3)]
      s0v = sb_v[ssl(0)]
      s1v = sb_v[ssl(1)]
      s2v = sb_v[ssl(2)]
      for j in range(L):
        iA = iav[j] * DW
        iB = ibv[j] * DW
        iC = icv[j] * DW
        iD = idv[j] * DW
        s0 = s0v[j]
        s1 = s1v[j]
        s2 = s2v[j]
        obase = (buf * B + m * L + j) * D
        accs = []
        for gg in range(4):
          go = gg * L
          ra = plsc.bitcast(ta_v[pl.ds(iA + go, L)], jnp.bfloat16)
          rb = plsc.bitcast(tb_v[pl.ds(iB + go, L)], jnp.bfloat16)
          rc = plsc.bitcast(tc_v[pl.ds(iC + go, L)], jnp.bfloat16)
          rd = plsc.bitcast(td_v[pl.ds(iD + go, L)], jnp.bfloat16)
          r = (ra + rb) + (rc + rd)
          e, o = plsc.unpack(r, format=plsc.PackFormat.INTERLEAVED)
          ke, ko = 2 * gg, 2 * gg + 1
          accs.append(e + (s0 * wcs[0][ke] + s1 * wcs[1][ke])
                      + s2 * wcs[2][ke])
          accs.append(o + (s0 * wcs[0][ko] + s1 * wcs[1][ko])
                      + s2 * wcs[2][ko])
        for k in range(8):
          o_v[pl.ds(obase + k * L, L)] = accs[k]

  @pl.when(main_batches > 0)
  def _():
    issue_inputs(0, base)

  @pl.when(main_batches > 1)
  def _():
    issue_inputs(1, base + B)

  def pair_body(p, _):
    for buf in range(2):
      g = p * 2 + buf
      off = base + g * B
      ob = pl.ds(buf * B * D, B * D)
      @pl.when(p > 0)
      def _():
        pltpu.make_async_copy(o_v.at[ob], out_h.at[pl.ds(base * D, B * D)],
                              osems[buf]).wait()
      drain_inputs(buf)
      compute_atoms(buf, B // L)
      pltpu.async_copy(o_v.at[ob], out_h.at[pl.ds(off * D, B * D)],
                       osems[buf])
      @pl.when(g + 2 < main_batches)
      def _():
        issue_inputs(buf, off + 2 * B)
    return 0

  lax.fori_loop(0, pairs, pair_body, 0)

  @pl.when(pairs > 0)
  def _():
    pltpu.make_async_copy(o_v.at[pl.ds(0, B * D)],
                          out_h.at[pl.ds(base * D, B * D)], osems[0]).wait()
    pltpu.make_async_copy(o_v.at[pl.ds(B * D, B * D)],
                          out_h.at[pl.ds(base * D, B * D)], osems[1]).wait()

  tail16 = (cnt - main_batches * B) // L

  def tail_body(t, _):
    off = base + main_batches * B + t * L
    src = pl.ds(off, L)
    pltpu.sync_copy(a_h.at[src], ai_v.at[pl.ds(0, L)])
    pltpu.sync_copy(d_h.at[src], di_v.at[pl.ds(0, L)])
    pltpu.sync_copy(c_h.at[src], ci_v.at[pl.ds(0, L)])
    pltpu.sync_copy(h_h.at[src], hi_v.at[pl.ds(0, L)])
    pltpu.sync_copy(n_h.at[src], ni_v.at[pl.ds(0, L)])
    pltpu.sync_copy(ch_h.at[src], chi_v.at[pl.ds(0, L)])
    pltpu.sync_copy(b0_h.at[src], bc_v.at[pl.ds(0, L)])
    pltpu.sync_copy(b1_h.at[src], bc_v.at[pl.ds(B, L)])
    pltpu.sync_copy(b2_h.at[src], bc_v.at[pl.ds(2 * B, L)])
    pltpu.sync_copy(b3_h.at[src], bc_v.at[pl.ds(3 * B, L)])
    pltpu.sync_copy(s0_h.at[src], sb_v.at[pl.ds(0, L)])
    pltpu.sync_copy(s1_h.at[src], sb_v.at[pl.ds(B, L)])
    pltpu.sync_copy(s2_h.at[src], sb_v.at[pl.ds(2 * B, L)])
    compute_atoms(0, 1)
    pltpu.sync_copy(o_v.at[pl.ds(0, L * D)], out_h.at[pl.ds(off * D, L * D)])
    return 0

  lax.fori_loop(0, tail16, tail_body, 0)


@functools.partial(jax.jit, static_argnames=("n_atoms",))
def _featurize(ta, tb, tc, td, wc, a, d, c, h, n, ch,
               b0, b1, b2, b3, s0, s1, s2, *, n_atoms):
  cnt_main = ((n_atoms + NW - 1) // NW + L - 1) // L * L
  mesh = plsc.VectorSubcoreMesh(core_axis_name="c", subcore_axis_name="s")
  body = functools.partial(_featurize_body, n_atoms=n_atoms,
                           cnt_main=cnt_main)
  return pl.kernel(
      body,
      out_type=jax.ShapeDtypeStruct((n_atoms * D,), jnp.float32),
      mesh=mesh,
      compiler_params=pltpu.CompilerParams(needs_layout_passes=False),
      scratch_types=[
          pltpu.VMEM((276 * DW,), jnp.int32),
          pltpu.VMEM((165 * DW,), jnp.int32),
          pltpu.VMEM((20 * DW,), jnp.int32),
          pltpu.VMEM((125 * DW,), jnp.int32),
          pltpu.VMEM((3, D), jnp.float32),
          pltpu.VMEM((2 * B,), jnp.int32),
          pltpu.VMEM((2 * B,), jnp.int32),
          pltpu.VMEM((2 * B,), jnp.int32),
          pltpu.VMEM((2 * B,), jnp.int32),
          pltpu.VMEM((2 * B,), jnp.int32),
          pltpu.VMEM((2 * B,), jnp.int32),
          pltpu.VMEM((8 * B,), jnp.int32),
          pltpu.VMEM((6 * B,), jnp.float32),
          pltpu.VMEM((2 * B * D,), jnp.float32),
          pltpu.SemaphoreType.DMA,
          pltpu.SemaphoreType.DMA,
          pltpu.SemaphoreType.DMA,
          pltpu.SemaphoreType.DMA,
      ],
  )(ta, tb, tc, td, wc, a, d, c, h, n, ch, b0, b1, b2, b3, s0, s1, s2)


def kernel(atom_type, degree, formal_charge, hybrid, num_h, chirality,
           bond_counts, scalar_base, atom_tab, deg_tab, charge_tab, hyb_tab,
           h_tab, chir_tab, bond_tab, W, b):
  n_atoms = atom_type.shape[0]
  cnt5 = jnp.arange(5, dtype=jnp.float32)
  gate = (cnt5 > 0).astype(jnp.float32)
  g = [bond_tab * gate[:, None] + (cnt5 / 4.0)[:, None] * W[:, 3 + j][None, :]
       for j in range(4)]
  ta = (atom_tab[:, None, :] + deg_tab[None, :, :]
        + b[None, None, :]).reshape(46 * 6, D)
  tb = (charge_tab[:, None, None, :] + hyb_tab[None, :, None, :]
        + h_tab[None, None, :, :]).reshape(11 * 3 * 5, D)
  tc = (chir_tab[:, None, :] + g[0][None, :, :]).reshape(4 * 5, D)
  td = (g[1][:, None, None, :] + g[2][None, :, None, :]
        + g[3][None, None, :, :]).reshape(5 * 5 * 5, D)
  perm = jnp.asarray(_PERM)

  def pack16(x):
    xb = x[:, perm].astype(jnp.bfloat16)
    return jax.lax.bitcast_convert_type(
        xb.reshape(x.shape[0], DW, 2), jnp.int32).reshape(-1)

  ta = pack16(ta)
  tb = pack16(tb)
  tc = pack16(tc)
  td = pack16(td)
  wc = jnp.transpose(W)[:3]
  a = atom_type.astype(jnp.int32)
  d = degree.astype(jnp.int32)
  c = formal_charge.astype(jnp.int32)
  h = hybrid.astype(jnp.int32)
  n = num_h.astype(jnp.int32)
  ch = chirality.astype(jnp.int32)
  bc = bond_counts.astype(jnp.int32)
  sb = scalar_base
  out = _featurize(ta, tb, tc, td, wc, a, d, c, h, n, ch,
                   bc[:, 0], bc[:, 1], bc[:, 2], bc[:, 3],
                   sb[:, 0], sb[:, 1], sb[:, 2], n_atoms=n_atoms)
  return out.reshape(n_atoms, D)


# speedup vs baseline: 35.2933x; 1.1259x over previous
import functools

import jax
import jax.numpy as jnp
import numpy as np
from jax import lax
from jax.experimental import pallas as pl
from jax.experimental.pallas import tpu as pltpu
from jax.experimental.pallas import tpu_sc as plsc

D = 128
DW = D // 2
B = 128
NC = 2
NS = 16
NW = NC * NS
L = 16

_PERM = np.concatenate([
    np.stack([np.arange(g * 32, g * 32 + 16),
              np.arange(g * 32 + 16, g * 32 + 32)], axis=1).reshape(-1)
    for g in range(4)])


def _featurize_body(ta_h, tb_h, tc_h, td_h, wc_h,
                    a_h, d_h, c_h, h_h, n_h, ch_h,
                    b0_h, b1_h, b2_h, b3_h, s0_h, s1_h, s2_h,
                    out_h,
                    ta_v, tb_v, tc_v, td_v, wc_v,
                    ai_v, di_v, ci_v, hi_v, ni_v, chi_v, bc_v, sb_v,
                    o_v,
                    is0, is1, os0, os1,
                    *, n_atoms, cnt_main):
  wid = lax.axis_index("s") * NC + lax.axis_index("c")
  base = wid * cnt_main
  cnt = jnp.minimum(cnt_main, n_atoms - base)
  trips = cnt // B
  pairs = trips // 2
  main_batches = pairs * 2
  isems = (is0, is1)
  osems = (os0, os1)

  pltpu.sync_copy(ta_h, ta_v)
  pltpu.sync_copy(tb_h, tb_v)
  pltpu.sync_copy(tc_h, tc_v)
  pltpu.sync_copy(td_h, td_v)
  pltpu.sync_copy(wc_h, wc_v)

  wcs = [[wc_v[j, pl.ds(k * L, L)] for k in range(8)] for j in range(3)]

  def issue_inputs(buf, off):
    ib = pl.ds(buf * B, B)
    src = pl.ds(off, B)
    pltpu.async_copy(a_h.at[src], ai_v.at[ib], isems[buf])
    pltpu.async_copy(d_h.at[src], di_v.at[ib], isems[buf])
    pltpu.async_copy(c_h.at[src], ci_v.at[ib], isems[buf])
    pltpu.async_copy(h_h.at[src], hi_v.at[ib], isems[buf])
    pltpu.async_copy(n_h.at[src], ni_v.at[ib], isems[buf])
    pltpu.async_copy(ch_h.at[src], chi_v.at[ib], isems[buf])
    pltpu.async_copy(b0_h.at[src], bc_v.at[pl.ds((buf * 4) * B, B)],
                     isems[buf])
    pltpu.async_copy(b1_h.at[src], bc_v.at[pl.ds((buf * 4 + 1) * B, B)],
                     isems[buf])
    pltpu.async_copy(b2_h.at[src], bc_v.at[pl.ds((buf * 4 + 2) * B, B)],
                     isems[buf])
    pltpu.async_copy(b3_h.at[src], bc_v.at[pl.ds((buf * 4 + 3) * B, B)],
                     isems[buf])
    pltpu.async_copy(s0_h.at[src], sb_v.at[pl.ds((buf * 3) * B, B)],
                     isems[buf])
    pltpu.async_copy(s1_h.at[src], sb_v.at[pl.ds((buf * 3 + 1) * B, B)],
                     isems[buf])
    pltpu.async_copy(s2_h.at[src], sb_v.at[pl.ds((buf * 3 + 2) * B, B)],
                     isems[buf])

  def drain_inputs(buf):
    ib = pl.ds(buf * B, B)
    src = pl.ds(0, B)
    pltpu.make_async_copy(a_h.at[src], ai_v.at[ib], isems[buf]).wait()
    pltpu.make_async_copy(d_h.at[src], di_v.at[ib], isems[buf]).wait()
    pltpu.make_async_copy(c_h.at[src], ci_v.at[ib], isems[buf]).wait()
    pltpu.make_async_copy(h_h.at[src], hi_v.at[ib], isems[buf]).wait()
    pltpu.make_async_copy(n_h.at[src], ni_v.at[ib], isems[buf]).wait()
    pltpu.make_async_copy(ch_h.at[src], chi_v.at[ib], isems[buf]).wait()
    pltpu.make_async_copy(b0_h.at[src], bc_v.at[pl.ds((buf * 4) * B, B)],
                          isems[buf]).wait()
    pltpu.make_async_copy(b1_h.at[src], bc_v.at[pl.ds((buf * 4 + 1) * B, B)],
                          isems[buf]).wait()
    pltpu.make_async_copy(b2_h.at[src], bc_v.at[pl.ds((buf * 4 + 2) * B, B)],
                          isems[buf]).wait()
    pltpu.make_async_copy(b3_h.at[src], bc_v.at[pl.ds((buf * 4 + 3) * B, B)],
                          isems[buf]).wait()
    pltpu.make_async_copy(s0_h.at[src], sb_v.at[pl.ds((buf * 3) * B, B)],
                          isems[buf]).wait()
    pltpu.make_async_copy(s1_h.at[src], sb_v.at[pl.ds((buf * 3 + 1) * B, B)],
                          isems[buf]).wait()
    pltpu.make_async_copy(s2_h.at[src], sb_v.at[pl.ds((buf * 3 + 2) * B, B)],
                          isems[buf]).wait()

  def compute_atoms(buf, nmac):
    @plsc.parallel_loop(0, nmac)
    def mbody(m):
      sl = pl.ds(buf * B + m * L, L)

      def bsl(j):
        return pl.ds((buf * 4 + j) * B + m * L, L)

      def ssl(j):
        return pl.ds((buf * 3 + j) * B + m * L, L)

      iav = (ai_v[sl] * 6 + di_v[sl]) * DW
      ibv = ((ci_v[sl] * 3 + hi_v[sl]) * 5 + ni_v[sl]) * DW
      icv = (chi_v[sl] * 5 + bc_v[bsl(0)]) * DW
      idv = ((bc_v[bsl(1)] * 5 + bc_v[bsl(2)]) * 5 + bc_v[bsl(3)]) * DW
      s0v = sb_v[ssl(0)]
      s1v = sb_v[ssl(1)]
      s2v = sb_v[ssl(2)]

      def atom_accs(j):
        iA = iav[j]
        iB = ibv[j]
        iC = icv[j]
        iD = idv[j]
        s0 = s0v[j]
        s1 = s1v[j]
        s2 = s2v[j]
        accs = []
        for gg in range(4):
          go = gg * L
          ra = plsc.bitcast(ta_v[pl.ds(iA + go, L)], jnp.bfloat16)
          rb = plsc.bitcast(tb_v[pl.ds(iB + go, L)], jnp.bfloat16)
          rc = plsc.bitcast(tc_v[pl.ds(iC + go, L)], jnp.bfloat16)
          rd = plsc.bitcast(td_v[pl.ds(iD + go, L)], jnp.bfloat16)
          r = (ra + rb) + (rc + rd)
          e, o = plsc.unpack(r, format=plsc.PackFormat.INTERLEAVED)
          ke, ko = 2 * gg, 2 * gg + 1
          accs.append(e + (s0 * wcs[0][ke] + s1 * wcs[1][ke])
                      + s2 * wcs[2][ke])
          accs.append(o + (s0 * wcs[0][ko] + s1 * wcs[1][ko])
                      + s2 * wcs[2][ko])
        return accs

      for j in range(0, L, 2):
        acc0 = atom_accs(j)
        acc1 = atom_accs(j + 1)
        ob0 = (buf * B + m * L + j) * D
        for k in range(8):
          o_v[pl.ds(ob0 + k * L, L)] = acc0[k]
        for k in range(8):
          o_v[pl.ds(ob0 + D + k * L, L)] = acc1[k]

  @pl.when(main_batches > 0)
  def _():
    issue_inputs(0, base)

  @pl.when(main_batches > 1)
  def _():
    issue_inputs(1, base + B)

  def pair_body(p, _):
    for buf in range(2):
      g = p * 2 + buf
      off = base + g * B
      ob = pl.ds(buf * B * D, B * D)
      @pl.when(p > 0)
      def _():
        pltpu.make_async_copy(o_v.at[ob], out_h.at[pl.ds(base * D, B * D)],
                              osems[buf]).wait()
      drain_inputs(buf)
      compute_atoms(buf, B // L)
      pltpu.async_copy(o_v.at[ob], out_h.at[pl.ds(off * D, B * D)],
                       osems[buf])
      @pl.when(g + 2 < main_batches)
      def _():
        issue_inputs(buf, off + 2 * B)
    return 0

  lax.fori_loop(0, pairs, pair_body, 0)

  @pl.when(pairs > 0)
  def _():
    pltpu.make_async_copy(o_v.at[pl.ds(0, B * D)],
                          out_h.at[pl.ds(base * D, B * D)], osems[0]).wait()
    pltpu.make_async_copy(o_v.at[pl.ds(B * D, B * D)],
                          out_h.at[pl.ds(base * D, B * D)], osems[1]).wait()

  tail16 = (cnt - main_batches * B) // L

  def tail_body(t, _):
    off = base + main_batches * B + t * L
    src = pl.ds(off, L)
    pltpu.sync_copy(a_h.at[src], ai_v.at[pl.ds(0, L)])
    pltpu.sync_copy(d_h.at[src], di_v.at[pl.ds(0, L)])
    pltpu.sync_copy(c_h.at[src], ci_v.at[pl.ds(0, L)])
    pltpu.sync_copy(h_h.at[src], hi_v.at[pl.ds(0, L)])
    pltpu.sync_copy(n_h.at[src], ni_v.at[pl.ds(0, L)])
    pltpu.sync_copy(ch_h.at[src], chi_v.at[pl.ds(0, L)])
    pltpu.sync_copy(b0_h.at[src], bc_v.at[pl.ds(0, L)])
    pltpu.sync_copy(b1_h.at[src], bc_v.at[pl.ds(B, L)])
    pltpu.sync_copy(b2_h.at[src], bc_v.at[pl.ds(2 * B, L)])
    pltpu.sync_copy(b3_h.at[src], bc_v.at[pl.ds(3 * B, L)])
    pltpu.sync_copy(s0_h.at[src], sb_v.at[pl.ds(0, L)])
    pltpu.sync_copy(s1_h.at[src], sb_v.at[pl.ds(B, L)])
    pltpu.sync_copy(s2_h.at[src], sb_v.at[pl.ds(2 * B, L)])
    compute_atoms(0, 1)
    pltpu.sync_copy(o_v.at[pl.ds(0, L * D)], out_h.at[pl.ds(off * D, L * D)])
    return 0

  lax.fori_loop(0, tail16, tail_body, 0)


@functools.partial(jax.jit, static_argnames=("n_atoms",))
def _featurize(ta, tb, tc, td, wc, a, d, c, h, n, ch,
               b0, b1, b2, b3, s0, s1, s2, *, n_atoms):
  cnt_main = ((n_atoms + NW - 1) // NW + L - 1) // L * L
  mesh = plsc.VectorSubcoreMesh(core_axis_name="c", subcore_axis_name="s")
  body = functools.partial(_featurize_body, n_atoms=n_atoms,
                           cnt_main=cnt_main)
  return pl.kernel(
      body,
      out_type=jax.ShapeDtypeStruct((n_atoms * D,), jnp.float32),
      mesh=mesh,
      compiler_params=pltpu.CompilerParams(needs_layout_passes=False),
      scratch_types=[
          pltpu.VMEM((276 * DW,), jnp.int32),
          pltpu.VMEM((165 * DW,), jnp.int32),
          pltpu.VMEM((20 * DW,), jnp.int32),
          pltpu.VMEM((125 * DW,), jnp.int32),
          pltpu.VMEM((3, D), jnp.float32),
          pltpu.VMEM((2 * B,), jnp.int32),
          pltpu.VMEM((2 * B,), jnp.int32),
          pltpu.VMEM((2 * B,), jnp.int32),
          pltpu.VMEM((2 * B,), jnp.int32),
          pltpu.VMEM((2 * B,), jnp.int32),
          pltpu.VMEM((2 * B,), jnp.int32),
          pltpu.VMEM((8 * B,), jnp.int32),
          pltpu.VMEM((6 * B,), jnp.float32),
          pltpu.VMEM((2 * B * D,), jnp.float32),
          pltpu.SemaphoreType.DMA,
          pltpu.SemaphoreType.DMA,
          pltpu.SemaphoreType.DMA,
          pltpu.SemaphoreType.DMA,
      ],
  )(ta, tb, tc, td, wc, a, d, c, h, n, ch, b0, b1, b2, b3, s0, s1, s2)


def kernel(atom_type, degree, formal_charge, hybrid, num_h, chirality,
           bond_counts, scalar_base, atom_tab, deg_tab, charge_tab, hyb_tab,
           h_tab, chir_tab, bond_tab, W, b):
  n_atoms = atom_type.shape[0]
  cnt5 = jnp.arange(5, dtype=jnp.float32)
  gate = (cnt5 > 0).astype(jnp.float32)
  g = [bond_tab * gate[:, None] + (cnt5 / 4.0)[:, None] * W[:, 3 + j][None, :]
       for j in range(4)]
  ta = (atom_tab[:, None, :] + deg_tab[None, :, :]
        + b[None, None, :]).reshape(46 * 6, D)
  tb = (charge_tab[:, None, None, :] + hyb_tab[None, :, None, :]
        + h_tab[None, None, :, :]).reshape(11 * 3 * 5, D)
  tc = (chir_tab[:, None, :] + g[0][None, :, :]).reshape(4 * 5, D)
  td = (g[1][:, None, None, :] + g[2][None, :, None, :]
        + g[3][None, None, :, :]).reshape(5 * 5 * 5, D)
  perm = jnp.asarray(_PERM)

  def pack16(x):
    xb = x[:, perm].astype(jnp.bfloat16)
    return jax.lax.bitcast_convert_type(
        xb.reshape(x.shape[0], DW, 2), jnp.int32).reshape(-1)

  ta = pack16(ta)
  tb = pack16(tb)
  tc = pack16(tc)
  td = pack16(td)
  wc = jnp.transpose(W)[:3]
  a = atom_type.astype(jnp.int32)
  d = degree.astype(jnp.int32)
  c = formal_charge.astype(jnp.int32)
  h = hybrid.astype(jnp.int32)
  n = num_h.astype(jnp.int32)
  ch = chirality.astype(jnp.int32)
  bc = bond_counts.astype(jnp.int32)
  sb = scalar_base
  out = _featurize(ta, tb, tc, td, wc, a, d, c, h, n, ch,
                   bc[:, 0], bc[:, 1], bc[:, 2], bc[:, 3],
                   sb[:, 0], sb[:, 1], sb[:, 2], n_atoms=n_atoms)
  return out.reshape(n_atoms, D)


# speedup vs baseline: 35.6496x; 1.0101x over previous
import functools

import jax
import jax.numpy as jnp
import numpy as np
from jax import lax
from jax.experimental import pallas as pl
from jax.experimental.pallas import tpu as pltpu
from jax.experimental.pallas import tpu_sc as plsc

D = 128
DW = D // 2
B = 128
NC = 2
NS = 16
NW = NC * NS
L = 16

_PERM = np.concatenate([
    np.stack([np.arange(g * 32, g * 32 + 16),
              np.arange(g * 32 + 16, g * 32 + 32)], axis=1).reshape(-1)
    for g in range(4)])


def _featurize_body(ta_h, tb_h, tc_h, td_h, wc_h,
                    a_h, d_h, c_h, h_h, n_h, ch_h,
                    b0_h, b1_h, b2_h, b3_h, s0_h, s1_h, s2_h,
                    out_h,
                    ta_v, tb_v, tc_v, td_v, wc_v,
                    ai_v, di_v, ci_v, hi_v, ni_v, chi_v, bc_v, sb_v,
                    o_v,
                    is0, is1, os0, os1,
                    *, n_atoms, cnt_main):
  wid = lax.axis_index("s") * NC + lax.axis_index("c")
  base = wid * cnt_main
  cnt = jnp.minimum(cnt_main, n_atoms - base)
  trips = cnt // B
  pairs = trips // 2
  main_batches = pairs * 2
  isems = (is0, is1)
  osems = (os0, os1)

  pltpu.sync_copy(ta_h, ta_v)
  pltpu.sync_copy(tb_h, tb_v)
  pltpu.sync_copy(tc_h, tc_v)
  pltpu.sync_copy(td_h, td_v)
  pltpu.sync_copy(wc_h, wc_v)

  wcs = [[wc_v[j, pl.ds(k * L, L)] for k in range(8)] for j in range(3)]

  def issue_inputs(buf, off):
    ib = pl.ds(buf * B, B)
    src = pl.ds(off, B)
    pltpu.async_copy(a_h.at[src], ai_v.at[ib], isems[buf])
    pltpu.async_copy(d_h.at[src], di_v.at[ib], isems[buf])
    pltpu.async_copy(c_h.at[src], ci_v.at[ib], isems[buf])
    pltpu.async_copy(h_h.at[src], hi_v.at[ib], isems[buf])
    pltpu.async_copy(n_h.at[src], ni_v.at[ib], isems[buf])
    pltpu.async_copy(ch_h.at[src], chi_v.at[ib], isems[buf])
    pltpu.async_copy(b0_h.at[src], bc_v.at[pl.ds((buf * 4) * B, B)],
                     isems[buf])
    pltpu.async_copy(b1_h.at[src], bc_v.at[pl.ds((buf * 4 + 1) * B, B)],
                     isems[buf])
    pltpu.async_copy(b2_h.at[src], bc_v.at[pl.ds((buf * 4 + 2) * B, B)],
                     isems[buf])
    pltpu.async_copy(b3_h.at[src], bc_v.at[pl.ds((buf * 4 + 3) * B, B)],
                     isems[buf])
    pltpu.async_copy(s0_h.at[src], sb_v.at[pl.ds((buf * 3) * B, B)],
                     isems[buf])
    pltpu.async_copy(s1_h.at[src], sb_v.at[pl.ds((buf * 3 + 1) * B, B)],
                     isems[buf])
    pltpu.async_copy(s2_h.at[src], sb_v.at[pl.ds((buf * 3 + 2) * B, B)],
                     isems[buf])

  def drain_inputs(buf):
    ib = pl.ds(buf * B, B)
    src = pl.ds(0, B)
    pltpu.make_async_copy(a_h.at[src], ai_v.at[ib], isems[buf]).wait()
    pltpu.make_async_copy(d_h.at[src], di_v.at[ib], isems[buf]).wait()
    pltpu.make_async_copy(c_h.at[src], ci_v.at[ib], isems[buf]).wait()
    pltpu.make_async_copy(h_h.at[src], hi_v.at[ib], isems[buf]).wait()
    pltpu.make_async_copy(n_h.at[src], ni_v.at[ib], isems[buf]).wait()
    pltpu.make_async_copy(ch_h.at[src], chi_v.at[ib], isems[buf]).wait()
    pltpu.make_async_copy(b0_h.at[src], bc_v.at[pl.ds((buf * 4) * B, B)],
                          isems[buf]).wait()
    pltpu.make_async_copy(b1_h.at[src], bc_v.at[pl.ds((buf * 4 + 1) * B, B)],
                          isems[buf]).wait()
    pltpu.make_async_copy(b2_h.at[src], bc_v.at[pl.ds((buf * 4 + 2) * B, B)],
                          isems[buf]).wait()
    pltpu.make_async_copy(b3_h.at[src], bc_v.at[pl.ds((buf * 4 + 3) * B, B)],
                          isems[buf]).wait()
    pltpu.make_async_copy(s0_h.at[src], sb_v.at[pl.ds((buf * 3) * B, B)],
                          isems[buf]).wait()
    pltpu.make_async_copy(s1_h.at[src], sb_v.at[pl.ds((buf * 3 + 1) * B, B)],
                          isems[buf]).wait()
    pltpu.make_async_copy(s2_h.at[src], sb_v.at[pl.ds((buf * 3 + 2) * B, B)],
                          isems[buf]).wait()

  def compute_atoms(buf, nmac):
    @plsc.parallel_loop(0, nmac)
    def mbody(m):
      sl = pl.ds(buf * B + m * L, L)

      def bsl(j):
        return pl.ds((buf * 4 + j) * B + m * L, L)

      def ssl(j):
        return pl.ds((buf * 3 + j) * B + m * L, L)

      iav = (ai_v[sl] * 6 + di_v[sl]) * DW
      ibv = ((ci_v[sl] * 3 + hi_v[sl]) * 5 + ni_v[sl]) * DW
      icv = (chi_v[sl] * 5 + bc_v[bsl(0)]) * DW
      idv = ((bc_v[bsl(1)] * 5 + bc_v[bsl(2)]) * 5 + bc_v[bsl(3)]) * DW
      s0v = sb_v[ssl(0)]
      s1v = sb_v[ssl(1)]
      s2v = sb_v[ssl(2)]

      def atom_accs(j):
        iA = iav[j]
        iB = ibv[j]
        iC = icv[j]
        iD = idv[j]
        s0 = s0v[j]
        s1 = s1v[j]
        s2 = s2v[j]
        accs = []
        for gg in range(4):
          go = gg * L
          ra = plsc.bitcast(ta_v[pl.ds(iA + go, L)], jnp.bfloat16)
          rb = plsc.bitcast(tb_v[pl.ds(iB + go, L)], jnp.bfloat16)
          rc = plsc.bitcast(tc_v[pl.ds(iC + go, L)], jnp.bfloat16)
          rd = plsc.bitcast(td_v[pl.ds(iD + go, L)], jnp.bfloat16)
          r = (ra + rb) + (rc + rd)
          e, o = plsc.unpack(r, format=plsc.PackFormat.INTERLEAVED)
          ke, ko = 2 * gg, 2 * gg + 1
          accs.append(e + (s0 * wcs[0][ke] + s1 * wcs[1][ke])
                      + s2 * wcs[2][ke])
          accs.append(o + (s0 * wcs[0][ko] + s1 * wcs[1][ko])
                      + s2 * wcs[2][ko])
        return accs

      for j in range(0, L, 4):
        group = [atom_accs(j + u) for u in range(4)]
        ob0 = (buf * B + m * L + j) * D
        for u in range(4):
          for k in range(8):
            o_v[pl.ds(ob0 + u * D + k * L, L)] = group[u][k]

  @pl.when(main_batches > 0)
  def _():
    issue_inputs(0, base)

  @pl.when(main_batches > 1)
  def _():
    issue_inputs(1, base + B)

  def pair_body(p, _):
    for buf in range(2):
      g = p * 2 + buf
      off = base + g * B
      ob = pl.ds(buf * B * D, B * D)
      @pl.when(p > 0)
      def _():
        pltpu.make_async_copy(o_v.at[ob], out_h.at[pl.ds(base * D, B * D)],
                              osems[buf]).wait()
      drain_inputs(buf)
      compute_atoms(buf, B // L)
      pltpu.async_copy(o_v.at[ob], out_h.at[pl.ds(off * D, B * D)],
                       osems[buf])
      @pl.when(g + 2 < main_batches)
      def _():
        issue_inputs(buf, off + 2 * B)
    return 0

  lax.fori_loop(0, pairs, pair_body, 0)

  @pl.when(pairs > 0)
  def _():
    pltpu.make_async_copy(o_v.at[pl.ds(0, B * D)],
                          out_h.at[pl.ds(base * D, B * D)], osems[0]).wait()
    pltpu.make_async_copy(o_v.at[pl.ds(B * D, B * D)],
                          out_h.at[pl.ds(base * D, B * D)], osems[1]).wait()

  tail16 = (cnt - main_batches * B) // L

  def tail_body(t, _):
    off = base + main_batches * B + t * L
    src = pl.ds(off, L)
    pltpu.sync_copy(a_h.at[src], ai_v.at[pl.ds(0, L)])
    pltpu.sync_copy(d_h.at[src], di_v.at[pl.ds(0, L)])
    pltpu.sync_copy(c_h.at[src], ci_v.at[pl.ds(0, L)])
    pltpu.sync_copy(h_h.at[src], hi_v.at[pl.ds(0, L)])
    pltpu.sync_copy(n_h.at[src], ni_v.at[pl.ds(0, L)])
    pltpu.sync_copy(ch_h.at[src], chi_v.at[pl.ds(0, L)])
    pltpu.sync_copy(b0_h.at[src], bc_v.at[pl.ds(0, L)])
    pltpu.sync_copy(b1_h.at[src], bc_v.at[pl.ds(B, L)])
    pltpu.sync_copy(b2_h.at[src], bc_v.at[pl.ds(2 * B, L)])
    pltpu.sync_copy(b3_h.at[src], bc_v.at[pl.ds(3 * B, L)])
    pltpu.sync_copy(s0_h.at[src], sb_v.at[pl.ds(0, L)])
    pltpu.sync_copy(s1_h.at[src], sb_v.at[pl.ds(B, L)])
    pltpu.sync_copy(s2_h.at[src], sb_v.at[pl.ds(2 * B, L)])
    compute_atoms(0, 1)
    pltpu.sync_copy(o_v.at[pl.ds(0, L * D)], out_h.at[pl.ds(off * D, L * D)])
    return 0

  lax.fori_loop(0, tail16, tail_body, 0)


@functools.partial(jax.jit, static_argnames=("n_atoms",))
def _featurize(ta, tb, tc, td, wc, a, d, c, h, n, ch,
               b0, b1, b2, b3, s0, s1, s2, *, n_atoms):
  cnt_main = ((n_atoms + NW - 1) // NW + L - 1) // L * L
  mesh = plsc.VectorSubcoreMesh(core_axis_name="c", subcore_axis_name="s")
  body = functools.partial(_featurize_body, n_atoms=n_atoms,
                           cnt_main=cnt_main)
  return pl.kernel(
      body,
      out_type=jax.ShapeDtypeStruct((n_atoms * D,), jnp.float32),
      mesh=mesh,
      compiler_params=pltpu.CompilerParams(needs_layout_passes=False),
      scratch_types=[
          pltpu.VMEM((276 * DW,), jnp.int32),
          pltpu.VMEM((165 * DW,), jnp.int32),
          pltpu.VMEM((20 * DW,), jnp.int32),
          pltpu.VMEM((125 * DW,), jnp.int32),
          pltpu.VMEM((3, D), jnp.float32),
          pltpu.VMEM((2 * B,), jnp.int32),
          pltpu.VMEM((2 * B,), jnp.int32),
          pltpu.VMEM((2 * B,), jnp.int32),
          pltpu.VMEM((2 * B,), jnp.int32),
          pltpu.VMEM((2 * B,), jnp.int32),
          pltpu.VMEM((2 * B,), jnp.int32),
          pltpu.VMEM((8 * B,), jnp.int32),
          pltpu.VMEM((6 * B,), jnp.float32),
          pltpu.VMEM((2 * B * D,), jnp.float32),
          pltpu.SemaphoreType.DMA,
          pltpu.SemaphoreType.DMA,
          pltpu.SemaphoreType.DMA,
          pltpu.SemaphoreType.DMA,
      ],
  )(ta, tb, tc, td, wc, a, d, c, h, n, ch, b0, b1, b2, b3, s0, s1, s2)


def kernel(atom_type, degree, formal_charge, hybrid, num_h, chirality,
           bond_counts, scalar_base, atom_tab, deg_tab, charge_tab, hyb_tab,
           h_tab, chir_tab, bond_tab, W, b):
  n_atoms = atom_type.shape[0]
  cnt5 = jnp.arange(5, dtype=jnp.float32)
  gate = (cnt5 > 0).astype(jnp.float32)
  g = [bond_tab * gate[:, None] + (cnt5 / 4.0)[:, None] * W[:, 3 + j][None, :]
       for j in range(4)]
  ta = (atom_tab[:, None, :] + deg_tab[None, :, :]
        + b[None, None, :]).reshape(46 * 6, D)
  tb = (charge_tab[:, None, None, :] + hyb_tab[None, :, None, :]
        + h_tab[None, None, :, :]).reshape(11 * 3 * 5, D)
  tc = (chir_tab[:, None, :] + g[0][None, :, :]).reshape(4 * 5, D)
  td = (g[1][:, None, None, :] + g[2][None, :, None, :]
        + g[3][None, None, :, :]).reshape(5 * 5 * 5, D)
  perm = jnp.asarray(_PERM)

  def pack16(x):
    xb = x[:, perm].astype(jnp.bfloat16)
    return jax.lax.bitcast_convert_type(
        xb.reshape(x.shape[0], DW, 2), jnp.int32).reshape(-1)

  ta = pack16(ta)
  tb = pack16(tb)
  tc = pack16(tc)
  td = pack16(td)
  wc = jnp.transpose(W)[:3]
  a = atom_type.astype(jnp.int32)
  d = degree.astype(jnp.int32)
  c = formal_charge.astype(jnp.int32)
  h = hybrid.astype(jnp.int32)
  n = num_h.astype(jnp.int32)
  ch = chirality.astype(jnp.int32)
  bc = bond_counts.astype(jnp.int32)
  sb = scalar_base
  out = _featurize(ta, tb, tc, td, wc, a, d, c, h, n, ch,
                   bc[:, 0], bc[:, 1], bc[:, 2], bc[:, 3],
                   sb[:, 0], sb[:, 1], sb[:, 2], n_atoms=n_atoms)
  return out.reshape(n_atoms, D)
